# baseline (speedup 1.0000x reference)
import functools

import jax
import jax.numpy as jnp
from jax import lax
from jax.experimental import pallas as pl
from jax.experimental.pallas import tpu as pltpu
from jax.experimental.pallas import tpu_sc as plsc

N = 10000
N_PAD = 10240
E_PAD = 327680
CHUNK = 128
NC, NS = 2, 16
NCH_TOT = E_PAD // CHUNK
NCH_TILE = NCH_TOT // (NC * NS)
RPT = N_PAD // NS
KB = 16
KC_HEAVY = 80
KC_LIGHT = 80

_vector_mesh = plsc.VectorSubcoreMesh(core_axis_name="c", subcore_axis_name="s")


def _make_deg_kernel():
    @functools.partial(
        pl.kernel,
        out_type=jax.ShapeDtypeStruct((NC * N_PAD,), jnp.float32),
        mesh=_vector_mesh,
        scratch_types=[
            pltpu.VMEM_SHARED((N_PAD,), jnp.float32),
            pltpu.VMEM((NCH_TILE, CHUNK), jnp.int32),
            pltpu.VMEM((CHUNK,), jnp.float32),
        ],
    )
    def deg_kernel(dst_hbm, ones_hbm, z1_hbm, out_hbm, accum, idx_v, ones_v):
        c = lax.axis_index("c")
        s = lax.axis_index("s")
        wid = c * NS + s
        pltpu.sync_copy(ones_hbm, ones_v)
        pltpu.sync_copy(dst_hbm.at[pl.ds(wid * NCH_TILE, NCH_TILE)], idx_v)
        pltpu.sync_copy(z1_hbm, accum.at[pl.ds(s * RPT, RPT)])
        plsc.subcore_barrier()

        @pl.loop(0, NCH_TILE)
        def _(j):
            pltpu.sync_copy(ones_v, accum.at[idx_v.at[j]], add=True)

        plsc.subcore_barrier()
        pltpu.sync_copy(
            accum.at[pl.ds(s * RPT, RPT)],
            out_hbm.at[pl.ds(c * N_PAD + s * RPT, RPT)],
        )

    return deg_kernel


def _make_agg_kernel(D):
    @functools.partial(
        pl.kernel,
        out_type=jax.ShapeDtypeStruct((NC * N_PAD, D), jnp.float32),
        mesh=_vector_mesh,
        scratch_types=[
            pltpu.VMEM_SHARED((N_PAD, D), jnp.float32),
            pltpu.VMEM((KB, CHUNK), jnp.int32),
            pltpu.VMEM((KB, CHUNK), jnp.int32),
            pltpu.VMEM((2, CHUNK, D), jnp.float32),
            pltpu.SemaphoreType.DMA,
            pltpu.SemaphoreType.DMA,
        ],
    )
    def agg_kernel(src_hbm, dst_hbm, v_hbm, z_hbm, out_hbm,
                   accum, srcidx_v, dstidx_v, rows_v, sem0, sem1):
        c = lax.axis_index("c")
        s = lax.axis_index("s")
        nbl = jnp.where(c == 1, KC_HEAVY // KB, KC_LIGHT // KB)
        base = jnp.where(c == 1, s * KC_HEAVY, NS * KC_HEAVY + s * KC_LIGHT)
        pltpu.sync_copy(z_hbm, accum.at[pl.ds(s * RPT, RPT)])
        plsc.subcore_barrier()

        @pl.loop(0, nbl)
        def _(bl):
            pltpu.sync_copy(src_hbm.at[pl.ds(base + bl * KB, KB)], srcidx_v)
            pltpu.sync_copy(dst_hbm.at[pl.ds(base + bl * KB, KB)], dstidx_v)
            pltpu.async_copy(v_hbm.at[srcidx_v.at[0]], rows_v.at[0], sem0)

            @pl.loop(0, KB, step=2)
            def _(j):
                pltpu.async_copy(v_hbm.at[srcidx_v.at[j + 1]], rows_v.at[1], sem1)
                pltpu.make_async_copy(
                    v_hbm.at[srcidx_v.at[j]], rows_v.at[0], sem0).wait()
                pltpu.sync_copy(rows_v.at[0], accum.at[dstidx_v.at[j]], add=True)

                @pl.when(j + 2 < KB)
                def _():
                    pltpu.async_copy(v_hbm.at[srcidx_v.at[j + 2]], rows_v.at[0], sem0)

                pltpu.make_async_copy(
                    v_hbm.at[srcidx_v.at[j + 1]], rows_v.at[1], sem1).wait()
                pltpu.sync_copy(rows_v.at[1], accum.at[dstidx_v.at[j + 1]], add=True)

        plsc.subcore_barrier()
        pltpu.sync_copy(
            accum.at[pl.ds(s * RPT, RPT)],
            out_hbm.at[pl.ds(c * N_PAD + s * RPT, RPT)],
        )

    return agg_kernel


BR = 1024


def _scale_body(hist_ref, x_ref, xs_ref, dinv_ref):
    deg = hist_ref[0] + hist_ref[1] + 1.0
    dinv = lax.rsqrt(deg)
    dinv_ref[...] = dinv
    xs_ref[...] = dinv * x_ref[...]


def _dense_body(aggp_ref, xs_ref, dinv_ref, w1_ref, b1_ref, w2_ref, xs2_ref):
    dinv = dinv_ref[...]
    t = dinv * (aggp_ref[0] + aggp_ref[1] + xs_ref[...])
    h = jnp.dot(t, w1_ref[...], preferred_element_type=jnp.float32) + b1_ref[...]
    h = jnp.maximum(h, 0.0)
    p2 = jnp.dot(h, w2_ref[...], preferred_element_type=jnp.float32)
    xs2_ref[...] = dinv * p2


def _finish_body(agg2_ref, xs2_ref, dinv_ref, b2_ref, o_ref):
    o_ref[...] = (
        dinv_ref[...] * (agg2_ref[0] + agg2_ref[1] + xs2_ref[...]) + b2_ref[...]
    )


def _tc_scale(hist3, x_pad):
    return pl.pallas_call(
        _scale_body,
        grid=(N_PAD // BR,),
        in_specs=[
            pl.BlockSpec((2, BR, 1), lambda i: (0, i, 0)),
            pl.BlockSpec((BR, 128), lambda i: (i, 0)),
        ],
        out_specs=[
            pl.BlockSpec((BR, 128), lambda i: (i, 0)),
            pl.BlockSpec((BR, 1), lambda i: (i, 0)),
        ],
        out_shape=[
            jax.ShapeDtypeStruct((N_PAD, 128), jnp.float32),
            jax.ShapeDtypeStruct((N_PAD, 1), jnp.float32),
        ],
    )(hist3, x_pad)


def _tc_dense(aggp, xs, dinv, W1, b1r, W2p):
    return pl.pallas_call(
        _dense_body,
        grid=(N_PAD // BR,),
        in_specs=[
            pl.BlockSpec((2, BR, 128), lambda i: (0, i, 0)),
            pl.BlockSpec((BR, 128), lambda i: (i, 0)),
            pl.BlockSpec((BR, 1), lambda i: (i, 0)),
            pl.BlockSpec((128, 256), lambda i: (0, 0)),
            pl.BlockSpec((1, 256), lambda i: (0, 0)),
            pl.BlockSpec((256, 128), lambda i: (0, 0)),
        ],
        out_specs=pl.BlockSpec((BR, 128), lambda i: (i, 0)),
        out_shape=jax.ShapeDtypeStruct((N_PAD, 128), jnp.float32),
    )(aggp, xs, dinv, W1, b1r, W2p)


def _tc_finish(agg2p, xs2, dinv, b2r):
    return pl.pallas_call(
        _finish_body,
        grid=(N_PAD // BR,),
        in_specs=[
            pl.BlockSpec((2, BR, 128), lambda i: (0, i, 0)),
            pl.BlockSpec((BR, 128), lambda i: (i, 0)),
            pl.BlockSpec((BR, 1), lambda i: (i, 0)),
            pl.BlockSpec((1, 128), lambda i: (0, 0)),
        ],
        out_specs=pl.BlockSpec((BR, 128), lambda i: (i, 0)),
        out_shape=jax.ShapeDtypeStruct((N_PAD, 128), jnp.float32),
    )(agg2p, xs2, dinv, b2r)


def kernel(x, edge_index, W1, b1, Wc, bc, Wk, bk):
    E = edge_index.shape[1]
    ei = edge_index.astype(jnp.int32)
    pad_src = jnp.full((E_PAD - E,), N, jnp.int32)
    pad_dst = N + (jnp.arange(E_PAD - E, dtype=jnp.int32) % (N_PAD - N))
    src2d = jnp.concatenate([ei[0], pad_src]).reshape(NCH_TOT, CHUNK)
    dst2d = jnp.concatenate([ei[1], pad_dst]).reshape(NCH_TOT, CHUNK)
    x_pad = jnp.pad(x, ((0, N_PAD - N), (0, 0)))

    ones128 = jnp.ones((CHUNK,), jnp.float32)
    z1 = jnp.zeros((RPT,), jnp.float32)
    z128 = jnp.zeros((RPT, 128), jnp.float32)

    hist = _make_deg_kernel()(dst2d, ones128, z1)
    hist3 = hist.reshape(NC, N_PAD, 1)
    xs, dinv = _tc_scale(hist3, x_pad)

    aggp = _make_agg_kernel(128)(src2d, dst2d, xs, z128).reshape(NC, N_PAD, 128)

    W2p = jnp.concatenate([Wc, Wk, jnp.zeros((256, 72), jnp.float32)], axis=1)
    b1r = b1.reshape(1, 256)
    xs2 = _tc_dense(aggp, xs, dinv, W1, b1r, W2p)

    agg2p = _make_agg_kernel(128)(src2d, dst2d, xs2, z128).reshape(NC, N_PAD, 128)

    b2r = jnp.concatenate([bc, bk, jnp.zeros((72,), jnp.float32)]).reshape(1, 128)
    outp = _tc_finish(agg2p, xs2, dinv, b2r)
    return outp[:N, :40], outp[:N, 40:56]


# speedup vs baseline: 2.7632x; 2.7632x over previous
import functools

import jax
import jax.numpy as jnp
from jax import lax
from jax.experimental import pallas as pl
from jax.experimental.pallas import tpu as pltpu
from jax.experimental.pallas import tpu_sc as plsc

N = 10000
N_PAD = 10240
E_PAD = 327680
CHUNK = 128
NC, NS = 2, 16
NCH_TOT = E_PAD // CHUNK
NCH_TILE = NCH_TOT // NS
NCH_TILE_E = NCH_TOT // (NC * NS)
RPT = N_PAD // NS
KB = 16

_vector_mesh = plsc.VectorSubcoreMesh(core_axis_name="c", subcore_axis_name="s")


def _make_deg_kernel():
    @functools.partial(
        pl.kernel,
        out_type=jax.ShapeDtypeStruct((NC * N_PAD,), jnp.float32),
        mesh=_vector_mesh,
        scratch_types=[
            pltpu.VMEM_SHARED((N_PAD,), jnp.float32),
            pltpu.VMEM((NCH_TILE_E, CHUNK), jnp.int32),
            pltpu.VMEM((CHUNK,), jnp.float32),
        ],
    )
    def deg_kernel(dst_hbm, ones_hbm, z1_hbm, out_hbm, accum, idx_v, ones_v):
        c = lax.axis_index("c")
        s = lax.axis_index("s")
        wid = c * NS + s
        pltpu.sync_copy(ones_hbm, ones_v)
        pltpu.sync_copy(dst_hbm.at[pl.ds(wid * NCH_TILE_E, NCH_TILE_E)], idx_v)
        pltpu.sync_copy(z1_hbm, accum.at[pl.ds(s * RPT, RPT)])
        plsc.subcore_barrier()

        @pl.loop(0, NCH_TILE_E)
        def _(j):
            pltpu.sync_copy(ones_v, accum.at[idx_v.at[j]], add=True)

        plsc.subcore_barrier()
        pltpu.sync_copy(
            accum.at[pl.ds(s * RPT, RPT)],
            out_hbm.at[pl.ds(c * N_PAD + s * RPT, RPT)],
        )

    return deg_kernel


def _make_agg_kernel(D):
    @functools.partial(
        pl.kernel,
        out_type=jax.ShapeDtypeStruct((NC * N_PAD, D), jnp.float32),
        mesh=_vector_mesh,
        scratch_types=[
            pltpu.VMEM_SHARED((N_PAD, D), jnp.float32),
            pltpu.VMEM_SHARED((N_PAD, D), jnp.float32),
            pltpu.VMEM((KB, CHUNK), jnp.int32),
            pltpu.VMEM((KB, CHUNK), jnp.int32),
            pltpu.VMEM((2, CHUNK, D), jnp.float32),
            pltpu.SemaphoreType.DMA,
            pltpu.SemaphoreType.DMA,
        ],
    )
    def agg_kernel(src_hbm, dst_hbm, va_hbm, vb_hbm, z_hbm, out_hbm,
                   accum, vstage, srcidx_v, dstidx_v, rows_v, sem0, sem1):
        c = lax.axis_index("c")
        s = lax.axis_index("s")
        rslc = pl.ds(s * RPT, RPT)

        @pl.when(c == 0)
        def _():
            pltpu.sync_copy(va_hbm.at[rslc], vstage.at[rslc])

        @pl.when(c == 1)
        def _():
            pltpu.sync_copy(vb_hbm.at[rslc], vstage.at[rslc])

        pltpu.sync_copy(z_hbm, accum.at[rslc])
        plsc.subcore_barrier()

        @pl.loop(0, NCH_TILE // KB)
        def _(bl):
            base = s * NCH_TILE + bl * KB
            pltpu.sync_copy(src_hbm.at[pl.ds(base, KB)], srcidx_v)
            pltpu.sync_copy(dst_hbm.at[pl.ds(base, KB)], dstidx_v)
            pltpu.async_copy(vstage.at[srcidx_v.at[0]], rows_v.at[0], sem0)

            @pl.loop(0, KB, step=2)
            def _(j):
                pltpu.async_copy(vstage.at[srcidx_v.at[j + 1]], rows_v.at[1], sem1)
                pltpu.make_async_copy(
                    vstage.at[srcidx_v.at[j]], rows_v.at[0], sem0).wait()
                pltpu.sync_copy(rows_v.at[0], accum.at[dstidx_v.at[j]], add=True)

                @pl.when(j + 2 < KB)
                def _():
                    pltpu.async_copy(
                        vstage.at[srcidx_v.at[j + 2]], rows_v.at[0], sem0)

                pltpu.make_async_copy(
                    vstage.at[srcidx_v.at[j + 1]], rows_v.at[1], sem1).wait()
                pltpu.sync_copy(rows_v.at[1], accum.at[dstidx_v.at[j + 1]], add=True)

        plsc.subcore_barrier()
        pltpu.sync_copy(accum.at[rslc], out_hbm.at[pl.ds(c * N_PAD + s * RPT, RPT)])

    return agg_kernel


BR = 1024


def _scale_body(hist_ref, x_ref, xsa_ref, xsb_ref, dinv_ref):
    deg = hist_ref[0] + hist_ref[1] + 1.0
    dinv = lax.rsqrt(deg)
    dinv_ref[...] = dinv
    xs = dinv * x_ref[...]
    xsa_ref[...] = xs[:, :64]
    xsb_ref[...] = xs[:, 64:]


def _dense_body(aggp_ref, xsa_ref, xsb_ref, dinv_ref, w1_ref, b1_ref, w2_ref,
                xs2a_ref, xs2b_ref):
    dinv = dinv_ref[...]
    xs = jnp.concatenate([xsa_ref[...], xsb_ref[...]], axis=1)
    agg = jnp.concatenate([aggp_ref[0], aggp_ref[1]], axis=1)
    t = dinv * (agg + xs)
    h = jnp.dot(t, w1_ref[...], preferred_element_type=jnp.float32) + b1_ref[...]
    h = jnp.maximum(h, 0.0)
    p2 = jnp.dot(h, w2_ref[...], preferred_element_type=jnp.float32)
    xs2 = dinv * p2
    xs2a_ref[...] = xs2[:, :32]
    xs2b_ref[...] = xs2[:, 32:]


def _finish_body(agg2_ref, xs2a_ref, xs2b_ref, dinv_ref, b2_ref, o_ref):
    xs2 = jnp.concatenate([xs2a_ref[...], xs2b_ref[...]], axis=1)
    agg2 = jnp.concatenate([agg2_ref[0], agg2_ref[1]], axis=1)
    o_ref[...] = dinv_ref[...] * (agg2 + xs2) + b2_ref[...]


def _tc_scale(hist3, x_pad):
    return pl.pallas_call(
        _scale_body,
        grid=(N_PAD // BR,),
        in_specs=[
            pl.BlockSpec((2, BR, 1), lambda i: (0, i, 0)),
            pl.BlockSpec((BR, 128), lambda i: (i, 0)),
        ],
        out_specs=[
            pl.BlockSpec((BR, 64), lambda i: (i, 0)),
            pl.BlockSpec((BR, 64), lambda i: (i, 0)),
            pl.BlockSpec((BR, 1), lambda i: (i, 0)),
        ],
        out_shape=[
            jax.ShapeDtypeStruct((N_PAD, 64), jnp.float32),
            jax.ShapeDtypeStruct((N_PAD, 64), jnp.float32),
            jax.ShapeDtypeStruct((N_PAD, 1), jnp.float32),
        ],
    )(hist3, x_pad)


def _tc_dense(aggp, xsa, xsb, dinv, W1, b1r, W2p):
    return pl.pallas_call(
        _dense_body,
        grid=(N_PAD // BR,),
        in_specs=[
            pl.BlockSpec((2, BR, 64), lambda i: (0, i, 0)),
            pl.BlockSpec((BR, 64), lambda i: (i, 0)),
            pl.BlockSpec((BR, 64), lambda i: (i, 0)),
            pl.BlockSpec((BR, 1), lambda i: (i, 0)),
            pl.BlockSpec((128, 256), lambda i: (0, 0)),
            pl.BlockSpec((1, 256), lambda i: (0, 0)),
            pl.BlockSpec((256, 64), lambda i: (0, 0)),
        ],
        out_specs=[
            pl.BlockSpec((BR, 32), lambda i: (i, 0)),
            pl.BlockSpec((BR, 32), lambda i: (i, 0)),
        ],
        out_shape=[
            jax.ShapeDtypeStruct((N_PAD, 32), jnp.float32),
            jax.ShapeDtypeStruct((N_PAD, 32), jnp.float32),
        ],
    )(aggp, xsa, xsb, dinv, W1, b1r, W2p)


def _tc_finish(agg2p, xs2a, xs2b, dinv, b2r):
    return pl.pallas_call(
        _finish_body,
        grid=(N_PAD // BR,),
        in_specs=[
            pl.BlockSpec((2, BR, 32), lambda i: (0, i, 0)),
            pl.BlockSpec((BR, 32), lambda i: (i, 0)),
            pl.BlockSpec((BR, 32), lambda i: (i, 0)),
            pl.BlockSpec((BR, 1), lambda i: (i, 0)),
            pl.BlockSpec((1, 64), lambda i: (0, 0)),
        ],
        out_specs=pl.BlockSpec((BR, 64), lambda i: (i, 0)),
        out_shape=jax.ShapeDtypeStruct((N_PAD, 64), jnp.float32),
    )(agg2p, xs2a, xs2b, dinv, b2r)


def kernel(x, edge_index, W1, b1, Wc, bc, Wk, bk):
    E = edge_index.shape[1]
    ei = edge_index.astype(jnp.int32)
    pad_src = jnp.full((E_PAD - E,), N, jnp.int32)
    pad_dst = N + (jnp.arange(E_PAD - E, dtype=jnp.int32) % (N_PAD - N))
    src2d = jnp.concatenate([ei[0], pad_src]).reshape(NCH_TOT, CHUNK)
    dst2d = jnp.concatenate([ei[1], pad_dst]).reshape(NCH_TOT, CHUNK)
    x_pad = jnp.pad(x, ((0, N_PAD - N), (0, 0)))

    ones128 = jnp.ones((CHUNK,), jnp.float32)
    z1 = jnp.zeros((RPT,), jnp.float32)
    z64 = jnp.zeros((RPT, 64), jnp.float32)
    z32 = jnp.zeros((RPT, 32), jnp.float32)

    hist = _make_deg_kernel()(dst2d, ones128, z1)
    hist3 = hist.reshape(NC, N_PAD, 1)
    xsa, xsb, dinv = _tc_scale(hist3, x_pad)

    aggp = _make_agg_kernel(64)(src2d, dst2d, xsa, xsb, z64).reshape(NC, N_PAD, 64)

    W2p = jnp.concatenate([Wc, Wk, jnp.zeros((256, 8), jnp.float32)], axis=1)
    b1r = b1.reshape(1, 256)
    xs2a, xs2b = _tc_dense(aggp, xsa, xsb, dinv, W1, b1r, W2p)

    agg2p = _make_agg_kernel(32)(src2d, dst2d, xs2a, xs2b, z32).reshape(NC, N_PAD, 32)

    b2r = jnp.concatenate([bc, bk, jnp.zeros((8,), jnp.float32)]).reshape(1, 64)
    outp = _tc_finish(agg2p, xs2a, xs2b, dinv, b2r)
    return outp[:N, :40], outp[:N, 40:56]
